# Initial kernel scaffold; baseline (speedup 1.0000x reference)
import jax
import jax.numpy as jnp
from jax.experimental import pallas as pl


def kernel(x, edge_index_distance, edge_weight_distance, E1, E2, W_agcn, b_agcn, W_cheb, b_cheb, Wf_a, bf_a, Wf_g, bf_g):
    raise NotImplementedError("write your pallas kernel here")


# speedup vs baseline: 5.8114x; 5.8114x over previous
import functools

import jax
import jax.numpy as jnp
from jax import lax
from jax.experimental import pallas as pl
from jax.experimental.pallas import tpu as pltpu
from jax.experimental.pallas import tpu_sc as plsc

NC = 2
NS = 16
LN = 16
CH = 128


def _rsqrt_newton(d):
    i = lax.bitcast_convert_type(d, jnp.int32)
    i = jnp.int32(0x5F3759DF) - (i >> 1)
    y = lax.bitcast_convert_type(i, jnp.float32)
    for _ in range(3):
        y = y * (1.5 - 0.5 * d * y * y)
    return y


def _make_norm_kernel(N, EPAD):
    NP = ((N + (LN * NS) - 1) // (LN * NS)) * (LN * NS)
    NPT = NP // NS
    EPT = EPAD // NS
    EPT2 = EPAD // (NS * NC)
    NCH = EPT // CH
    NCH2 = EPT2 // CH
    mesh = plsc.VectorSubcoreMesh(core_axis_name="c", subcore_axis_name="s")

    def body(row_hbm, col_hbm, w_hbm, norm_hbm,
             deg_sh, dinv_sh,
             zbuf, ist, wst, nodebuf, dloc, nbuf, sem):
        cid = lax.axis_index("c")
        sid = lax.axis_index("s")
        my0 = sid * NPT

        for i in range(NPT // LN):
            zbuf[pl.ds(i * LN, LN)] = jnp.zeros((LN,), jnp.float32)
        pltpu.sync_copy(zbuf, deg_sh.at[pl.ds(my0, NPT)])
        plsc.subcore_barrier()

        @pl.loop(0, NCH)
        def _deg(c):
            base = sid * EPT + c * CH
            pltpu.sync_copy(row_hbm.at[pl.ds(base, CH)], ist.at[0])
            pltpu.sync_copy(w_hbm.at[pl.ds(base, CH)], wst)
            pltpu.sync_copy(wst, deg_sh.at[ist.at[0]], add=True)

        plsc.subcore_barrier()

        pltpu.sync_copy(deg_sh.at[pl.ds(my0, NPT)], nodebuf)
        for i in range(NPT // LN):
            d = nodebuf[pl.ds(i * LN, LN)]
            y = _rsqrt_newton(d)
            nodebuf[pl.ds(i * LN, LN)] = jnp.where(d > 0.0, y, 0.0)
        pltpu.sync_copy(nodebuf, dinv_sh.at[pl.ds(my0, NPT)])
        plsc.subcore_barrier()

        pltpu.sync_copy(dinv_sh, dloc)

        @pl.loop(0, NCH2)
        def _norm(c):
            base = cid * (EPAD // NC) + sid * EPT2 + c * CH
            pltpu.sync_copy(row_hbm.at[pl.ds(base, CH)], ist.at[0])
            pltpu.sync_copy(col_hbm.at[pl.ds(base, CH)], ist.at[1])
            pltpu.sync_copy(w_hbm.at[pl.ds(base, CH)], wst)
            for g in range(CH // LN):
                rv = ist[0, pl.ds(g * LN, LN)]
                cv = ist[1, pl.ds(g * LN, LN)]
                wv = wst[pl.ds(g * LN, LN)]
                dr = plsc.load_gather(dloc, [rv])
                dc = plsc.load_gather(dloc, [cv])
                nbuf[pl.ds(g * LN, LN)] = -(wv * dr) * dc
            pltpu.sync_copy(nbuf, norm_hbm.at[pl.ds(base, CH)])

    kern = pl.kernel(
        body,
        out_type=jax.ShapeDtypeStruct((EPAD,), jnp.float32),
        mesh=mesh,
        compiler_params=pltpu.CompilerParams(needs_layout_passes=False),
        scratch_types=[
            pltpu.VMEM_SHARED((NP,), jnp.float32),
            pltpu.VMEM_SHARED((NP,), jnp.float32),
            pltpu.VMEM((NPT,), jnp.float32),
            pltpu.VMEM((2, CH), jnp.int32),
            pltpu.VMEM((CH,), jnp.float32),
            pltpu.VMEM((NPT,), jnp.float32),
            pltpu.VMEM((NP,), jnp.float32),
            pltpu.VMEM((CH,), jnp.float32),
            pltpu.SemaphoreType.DMA,
        ],
    )
    return kern


def _make_prop_kernel(N, D, M, EPAD, src_rows, stride, base_of_m):
    NP = ((N + (LN * NS) - 1) // (LN * NS)) * (LN * NS)
    NPT = NP // NS
    EPT = EPAD // NS
    NCH = EPT // CH
    MC = M // NC
    ZR = 128
    mesh = plsc.VectorSubcoreMesh(core_axis_name="c", subcore_axis_name="s")

    def body(src_hbm, row_hbm, col_hbm, norm_hbm, dst_hbm,
             acc_sh, zbuf, rowst, colst, gidx, nst, pay, sem):
        cid = lax.axis_index("c")
        sid = lax.axis_index("s")
        my0 = sid * NPT

        for i in range(ZR * D // LN):
            zbuf[(i * LN) // D, pl.ds((i * LN) % D, LN)] = jnp.zeros((LN,), jnp.float32)

        @pl.loop(0, MC)
        def _inst(inst):
            m = inst * NC + cid
            base_m = base_of_m(m)

            for z in range(NPT // ZR):
                pltpu.sync_copy(zbuf, acc_sh.at[pl.ds(my0 + z * ZR, ZR)])
            plsc.subcore_barrier()

            @pl.loop(0, NCH)
            def _chunk(c):
                base = sid * EPT + c * CH
                pltpu.sync_copy(row_hbm.at[pl.ds(base, CH)], rowst.at[0])
                pltpu.sync_copy(col_hbm.at[pl.ds(base, CH)], colst.at[0])
                pltpu.sync_copy(norm_hbm.at[pl.ds(base, CH)], nst)
                for g in range(CH // LN):
                    rv = rowst[0, pl.ds(g * LN, LN)]
                    gidx[0, pl.ds(g * LN, LN)] = rv * stride + base_m
                pltpu.async_copy(src_hbm.at[gidx.at[0]], pay, sem).wait()
                for g in range(CH // LN):
                    nv = nst[pl.ds(g * LN, LN)]
                    for i in range(LN):
                        e = g * LN + i
                        s = nv[i]
                        for j in range(D // LN):
                            v = pay[e, pl.ds(j * LN, LN)]
                            pay[e, pl.ds(j * LN, LN)] = v * s
                pltpu.sync_copy(pay, acc_sh.at[colst.at[0]], add=True)

            plsc.subcore_barrier()

            if N % NPT == 0:
                pltpu.sync_copy(acc_sh.at[pl.ds(my0, NPT)],
                                dst_hbm.at[m].at[pl.ds(my0, NPT)])
            else:
                nfull = N // NPT

                @pl.when(sid < nfull)
                def _full():
                    pltpu.sync_copy(acc_sh.at[pl.ds(my0, NPT)],
                                    dst_hbm.at[m].at[pl.ds(my0, NPT)])

                @pl.when(sid == nfull)
                def _tail():
                    rem = N - nfull * NPT
                    pltpu.sync_copy(acc_sh.at[pl.ds(my0, rem)],
                                    dst_hbm.at[m].at[pl.ds(my0, rem)])

    kern = pl.kernel(
        body,
        out_type=jax.ShapeDtypeStruct((M, N, D), jnp.float32),
        mesh=mesh,
        compiler_params=pltpu.CompilerParams(needs_layout_passes=False),
        scratch_types=[
            pltpu.VMEM_SHARED((NP, D), jnp.float32),
            pltpu.VMEM((ZR, D), jnp.float32),
            pltpu.VMEM((1, CH), jnp.int32),
            pltpu.VMEM((1, CH), jnp.int32),
            pltpu.VMEM((1, CH), jnp.int32),
            pltpu.VMEM((CH,), jnp.float32),
            pltpu.VMEM((CH, D), jnp.float32),
            pltpu.SemaphoreType.DMA,
        ],
    )
    return kern


def _tmp_body(x_ref, e2_ref, out_ref):
    nb = pl.program_id(1)

    @pl.when(nb == 0)
    def _():
        out_ref[...] = jnp.zeros_like(out_ref)

    xb = x_ref[0]
    e2b = e2_ref[...]
    out_ref[0] += lax.dot_general(e2b, xb, (((0,), (0,)), ((), ())),
                                  preferred_element_type=jnp.float32)


def _fused_body(x_ref, t1_ref, p2_ref, tmp_ref, e1_ref,
                wa_ref, ba_ref, wc_ref, bc_ref, wfa_ref, bfa_ref,
                wfg_ref, bfg_ref, out_ref):
    xb = x_ref[0]
    t1b = t1_ref[0]
    p2b = p2_ref[0]
    tmpm = tmp_ref[0]
    e1b = e1_ref[...]
    dot = functools.partial(jnp.dot, preferred_element_type=jnp.float32)

    h = jax.nn.relu(dot(e1b, tmpm))
    agcn = dot(h, wa_ref[...]) + ba_ref[0]
    w0 = wc_ref[0]
    w1 = wc_ref[1]
    w2 = wc_ref[2]
    cheb = (dot(xb, w0 - w2) + dot(t1b, w1) + 2.0 * dot(p2b, w2) + bc_ref[0])
    gate = jax.nn.sigmoid(dot(agcn, wfa_ref[...]) + bfa_ref[0]
                          + dot(cheb, wfg_ref[...]) + bfg_ref[0])
    out_ref[0] = gate * agcn + (1.0 - gate) * cheb


def kernel(x, edge_index_distance, edge_weight_distance, E1, E2,
           W_agcn, b_agcn, W_cheb, b_cheb, Wf_a, bf_a, Wf_g, bf_g):
    B, N, T, D = x.shape
    M = B * T
    H = W_agcn.shape[1]
    R = E1.shape[1]
    E = edge_weight_distance.shape[0]
    RP = ((R + 7) // 8) * 8

    EUNIT = NC * NS * CH
    EPAD = ((E + EUNIT - 1) // EUNIT) * EUNIT
    padn = EPAD - E
    row = edge_index_distance[0].astype(jnp.int32)
    col = edge_index_distance[1].astype(jnp.int32)
    w = edge_weight_distance.astype(jnp.float32)
    if padn:
        spread = jnp.arange(padn, dtype=jnp.int32) % N
        row = jnp.concatenate([row, spread])
        col = jnp.concatenate([col, spread])
        w = jnp.concatenate([w, jnp.zeros((padn,), jnp.float32)])

    norm = _make_norm_kernel(N, EPAD)(row, col, w)

    xflat = x.reshape(B * N * T, D)
    prop1 = _make_prop_kernel(
        N, D, M, EPAD, B * N * T, T,
        lambda m: (m // T) * (N * T) + (m % T))
    tx1 = prop1(xflat, row, col, norm)

    prop2 = _make_prop_kernel(
        N, D, M, EPAD, M * N, 1,
        lambda m: m * N)
    p2 = prop2(tx1.reshape(M * N, D), row, col, norm)

    PB = 1000
    NBLK = N // PB
    E2p = jnp.zeros((N, RP), jnp.float32).at[:, :R].set(E2.T)
    E1p = jnp.zeros((N, RP), jnp.float32).at[:, :R].set(E1)

    xv = x.reshape(B, N, T * D)
    tmp = pl.pallas_call(
        _tmp_body,
        grid=(M, NBLK),
        in_specs=[
            pl.BlockSpec((1, PB, D), lambda m, nb: (m // T, nb, m % T)),
            pl.BlockSpec((PB, RP), lambda m, nb: (nb, 0)),
        ],
        out_specs=pl.BlockSpec((1, RP, D), lambda m, nb: (m, 0, 0)),
        out_shape=jax.ShapeDtypeStruct((M, RP, D), jnp.float32),
    )(xv, E2p)

    full2 = lambda a, b: pl.BlockSpec((a, b), lambda m, nb: (0, 0))
    out = pl.pallas_call(
        _fused_body,
        grid=(M, NBLK),
        in_specs=[
            pl.BlockSpec((1, PB, D), lambda m, nb: (m // T, nb, m % T)),
            pl.BlockSpec((1, PB, D), lambda m, nb: (m, nb, 0)),
            pl.BlockSpec((1, PB, D), lambda m, nb: (m, nb, 0)),
            pl.BlockSpec((1, RP, D), lambda m, nb: (m, 0, 0)),
            pl.BlockSpec((PB, RP), lambda m, nb: (nb, 0)),
            full2(D, H),
            full2(1, H),
            pl.BlockSpec((3, D, H), lambda m, nb: (0, 0, 0)),
            full2(1, H),
            full2(H, H),
            full2(1, H),
            full2(H, H),
            full2(1, H),
        ],
        out_specs=pl.BlockSpec((1, PB, H), lambda m, nb: (m // T, nb, m % T)),
        out_shape=jax.ShapeDtypeStruct((B, N, T * H), jnp.float32),
    )(xv, tx1, p2, tmp, E1p,
      W_agcn, b_agcn.reshape(1, H), W_cheb, b_cheb.reshape(1, H),
      Wf_a, bf_a.reshape(1, H), Wf_g, bf_g.reshape(1, H))
    return out.reshape(B, N, T, H)


# speedup vs baseline: 13.8168x; 2.3775x over previous
import functools

import jax
import jax.numpy as jnp
from jax import lax
from jax.experimental import pallas as pl
from jax.experimental.pallas import tpu as pltpu
from jax.experimental.pallas import tpu_sc as plsc

NC = 2
NS = 16
LN = 16
CH = 128


def _rsqrt_newton(d):
    i = lax.bitcast_convert_type(d, jnp.int32)
    i = jnp.int32(0x5F3759DF) - (i >> 1)
    y = lax.bitcast_convert_type(i, jnp.float32)
    for _ in range(3):
        y = y * (1.5 - 0.5 * d * y * y)
    return y


def _make_norm_kernel(N, EPAD):
    NP = ((N + (LN * NS) - 1) // (LN * NS)) * (LN * NS)
    NPT = NP // NS
    EPT = EPAD // NS
    EPT2 = EPAD // (NS * NC)
    NCH = EPT // CH
    NCH2 = EPT2 // CH
    mesh = plsc.VectorSubcoreMesh(core_axis_name="c", subcore_axis_name="s")

    def body(row_hbm, col_hbm, w_hbm, norm_hbm,
             deg_sh, dinv_sh,
             zbuf, ist, wst, nodebuf, dloc, nbuf, sem):
        cid = lax.axis_index("c")
        sid = lax.axis_index("s")
        my0 = sid * NPT

        for i in range(NPT // LN):
            zbuf[pl.ds(i * LN, LN)] = jnp.zeros((LN,), jnp.float32)
        pltpu.sync_copy(zbuf, deg_sh.at[pl.ds(my0, NPT)])
        plsc.subcore_barrier()

        @pl.loop(0, NCH)
        def _deg(c):
            base = sid * EPT + c * CH
            pltpu.sync_copy(row_hbm.at[pl.ds(base, CH)], ist.at[0])
            pltpu.sync_copy(w_hbm.at[pl.ds(base, CH)], wst)
            pltpu.sync_copy(wst, deg_sh.at[ist.at[0]], add=True)

        plsc.subcore_barrier()

        pltpu.sync_copy(deg_sh.at[pl.ds(my0, NPT)], nodebuf)
        for i in range(NPT // LN):
            d = nodebuf[pl.ds(i * LN, LN)]
            y = _rsqrt_newton(d)
            nodebuf[pl.ds(i * LN, LN)] = jnp.where(d > 0.0, y, 0.0)
        pltpu.sync_copy(nodebuf, dinv_sh.at[pl.ds(my0, NPT)])
        plsc.subcore_barrier()

        pltpu.sync_copy(dinv_sh, dloc)

        @pl.loop(0, NCH2)
        def _norm(c):
            base = cid * (EPAD // NC) + sid * EPT2 + c * CH
            pltpu.sync_copy(row_hbm.at[pl.ds(base, CH)], ist.at[0])
            pltpu.sync_copy(col_hbm.at[pl.ds(base, CH)], ist.at[1])
            pltpu.sync_copy(w_hbm.at[pl.ds(base, CH)], wst)
            for g in range(CH // LN):
                rv = ist[0, pl.ds(g * LN, LN)]
                cv = ist[1, pl.ds(g * LN, LN)]
                wv = wst[pl.ds(g * LN, LN)]
                dr = plsc.load_gather(dloc, [rv])
                dc = plsc.load_gather(dloc, [cv])
                nbuf[pl.ds(g * LN, LN)] = -(wv * dr) * dc
            pltpu.sync_copy(nbuf, norm_hbm.at[pl.ds(base, CH)])

    kern = pl.kernel(
        body,
        out_type=jax.ShapeDtypeStruct((EPAD,), jnp.float32),
        mesh=mesh,
        compiler_params=pltpu.CompilerParams(needs_layout_passes=False),
        scratch_types=[
            pltpu.VMEM_SHARED((NP,), jnp.float32),
            pltpu.VMEM_SHARED((NP,), jnp.float32),
            pltpu.VMEM((NPT,), jnp.float32),
            pltpu.VMEM((2, CH), jnp.int32),
            pltpu.VMEM((CH,), jnp.float32),
            pltpu.VMEM((NPT,), jnp.float32),
            pltpu.VMEM((NP,), jnp.float32),
            pltpu.VMEM((CH,), jnp.float32),
            pltpu.SemaphoreType.DMA,
        ],
    )
    return kern


def _make_prop_kernel(N, D, M, EPAD, src_rows, stride, base_of_m):
    NPT = ((N // NS + 7) // 8) * 8
    NP = NPT * NS
    EPT = EPAD // NS
    NCH = EPT // CH
    MC = M // NC
    ZR = 64
    mesh = plsc.VectorSubcoreMesh(core_axis_name="c", subcore_axis_name="s")

    def body(src_hbm, pk_hbm, norm_hbm, dst_hbm,
             acc_sh, zbuf, pkst, nst, gidx, scidx,
             pay0, pay1, si0, si1, sp0, sp1, ss0, ss1):
        cid = lax.axis_index("c")
        sid = lax.axis_index("s")
        my0 = sid * NPT
        pays = (pay0, pay1)
        sis = (si0, si1)
        sps = (sp0, sp1)
        sss = (ss0, ss1)

        for i in range(ZR * D // LN):
            zbuf[(i * LN) // D, pl.ds((i * LN) % D, LN)] = jnp.zeros((LN,), jnp.float32)

        def idx_start(b, c):
            base = sid * EPT + c * CH
            pltpu.async_copy(pk_hbm.at[pl.ds(base, CH)], pkst.at[b], sis[b])
            pltpu.async_copy(norm_hbm.at[pl.ds(base, CH)], nst.at[b], sis[b])

        def idx_wait(b):
            pltpu.make_async_copy(pk_hbm.at[pl.ds(0, CH)], pkst.at[b], sis[b]).wait()
            pltpu.make_async_copy(norm_hbm.at[pl.ds(0, CH)], nst.at[b], sis[b]).wait()

        def gs_compute(b, base_m):
            for g in range(CH // LN):
                pv = pkst[b, pl.ds(g * LN, LN)]
                gidx[b, pl.ds(g * LN, LN)] = (pv >> 14) * stride + base_m
                scidx[b, pl.ds(g * LN, LN)] = pv & 16383

        def gather_start(b):
            pltpu.async_copy(src_hbm.at[gidx.at[b]], pays[b], sps[b])

        def gather_wait(b):
            pltpu.make_async_copy(src_hbm.at[gidx.at[b]], pays[b], sps[b]).wait()

        def scale(b):
            pay = pays[b]
            @pl.loop(0, CH // LN)
            def _grp(g):
                nv = nst[b, pl.ds(g * LN, LN)]
                for i in range(LN):
                    for j in range(D // LN):
                        v = pay[g * LN + i, pl.ds(j * LN, LN)]
                        pay[g * LN + i, pl.ds(j * LN, LN)] = v * nv[i]

        def scatter_start(b):
            pltpu.async_copy(pays[b], acc_sh.at[scidx.at[b]], sss[b], add=True)

        def scatter_wait(b):
            pltpu.make_async_copy(pays[b], acc_sh.at[scidx.at[b]], sss[b]).wait()

        @pl.loop(0, MC)
        def _inst(inst):
            m = inst * NC + cid
            base_m = base_of_m(m)

            for z in range(NPT // ZR):
                pltpu.sync_copy(zbuf, acc_sh.at[pl.ds(my0 + z * ZR, ZR)])
            if NPT % ZR:
                pltpu.sync_copy(zbuf.at[pl.ds(0, NPT % ZR)],
                                acc_sh.at[pl.ds(my0 + (NPT // ZR) * ZR, NPT % ZR)])
            plsc.subcore_barrier()

            idx_start(0, 0)
            idx_start(1, 1)
            idx_wait(0)
            gs_compute(0, base_m)
            gather_start(0)
            idx_wait(1)
            gs_compute(1, base_m)
            gather_start(1)
            gather_wait(0)
            scale(0)
            idx_start(0, 2)
            scatter_start(0)

            @pl.loop(0, (NCH - 2) // 2)
            def _pair(p):
                for b, cc in ((1, 2 * p + 1), (0, 2 * p + 2)):
                    nb = 1 - b
                    idx_wait(nb)
                    scatter_wait(nb)
                    gs_compute(nb, base_m)
                    gather_start(nb)
                    gather_wait(b)
                    scale(b)
                    if b == 1:
                        idx_start(b, 2 * p + 3)
                    else:
                        @pl.when(2 * p + 4 < NCH)
                        def _():
                            idx_start(0, 2 * p + 4)
                    scatter_start(b)

            gather_wait(1)
            scale(1)
            scatter_start(1)
            scatter_wait(0)
            scatter_wait(1)

            plsc.subcore_barrier()

            if N % NPT == 0:
                pltpu.sync_copy(acc_sh.at[pl.ds(my0, NPT)],
                                dst_hbm.at[m].at[pl.ds(my0, NPT)])
            else:
                nfull = N // NPT

                @pl.when(sid < nfull)
                def _full():
                    pltpu.sync_copy(acc_sh.at[pl.ds(my0, NPT)],
                                    dst_hbm.at[m].at[pl.ds(my0, NPT)])

                @pl.when(sid == nfull)
                def _tail():
                    rem = N - nfull * NPT
                    pltpu.sync_copy(acc_sh.at[pl.ds(my0, rem)],
                                    dst_hbm.at[m].at[pl.ds(my0, rem)])

    kern = pl.kernel(
        body,
        out_type=jax.ShapeDtypeStruct((M, N, D), jnp.float32),
        mesh=mesh,
        compiler_params=pltpu.CompilerParams(needs_layout_passes=False),
        scratch_types=[
            pltpu.VMEM_SHARED((NP, D), jnp.float32),
            pltpu.VMEM((ZR, D), jnp.float32),
            pltpu.VMEM((2, CH), jnp.int32),
            pltpu.VMEM((2, CH), jnp.float32),
            pltpu.VMEM((2, CH), jnp.int32),
            pltpu.VMEM((2, CH), jnp.int32),
            pltpu.VMEM((CH, D), jnp.float32),
            pltpu.VMEM((CH, D), jnp.float32),
            pltpu.SemaphoreType.DMA,
            pltpu.SemaphoreType.DMA,
            pltpu.SemaphoreType.DMA,
            pltpu.SemaphoreType.DMA,
            pltpu.SemaphoreType.DMA,
            pltpu.SemaphoreType.DMA,
        ],
    )
    return kern


def _tmp_body(x_ref, e2_ref, out_ref):
    nb = pl.program_id(1)

    @pl.when(nb == 0)
    def _():
        out_ref[...] = jnp.zeros_like(out_ref)

    xb = x_ref[0]
    e2b = e2_ref[...]
    out_ref[0] += lax.dot_general(e2b, xb, (((0,), (0,)), ((), ())),
                                  preferred_element_type=jnp.float32)


def _fused_body(x_ref, t1_ref, p2_ref, tmp_ref, e1_ref,
                wa_ref, ba_ref, wc_ref, bc_ref, wfa_ref, bfa_ref,
                wfg_ref, bfg_ref, out_ref):
    xb = x_ref[0]
    t1b = t1_ref[0]
    p2b = p2_ref[0]
    tmpm = tmp_ref[0]
    e1b = e1_ref[...]
    dot = functools.partial(jnp.dot, preferred_element_type=jnp.float32)

    h = jax.nn.relu(dot(e1b, tmpm))
    agcn = dot(h, wa_ref[...]) + ba_ref[0]
    w0 = wc_ref[0]
    w1 = wc_ref[1]
    w2 = wc_ref[2]
    cheb = (dot(xb, w0 - w2) + dot(t1b, w1) + 2.0 * dot(p2b, w2) + bc_ref[0])
    gate = jax.nn.sigmoid(dot(agcn, wfa_ref[...]) + bfa_ref[0]
                          + dot(cheb, wfg_ref[...]) + bfg_ref[0])
    out_ref[0] = gate * agcn + (1.0 - gate) * cheb


def kernel(x, edge_index_distance, edge_weight_distance, E1, E2,
           W_agcn, b_agcn, W_cheb, b_cheb, Wf_a, bf_a, Wf_g, bf_g):
    B, N, T, D = x.shape
    M = B * T
    H = W_agcn.shape[1]
    R = E1.shape[1]
    E = edge_weight_distance.shape[0]
    RP = ((R + 7) // 8) * 8

    EUNIT = NS * CH * 8
    EPAD = ((E + EUNIT - 1) // EUNIT) * EUNIT
    padn = EPAD - E
    row = edge_index_distance[0].astype(jnp.int32)
    col = edge_index_distance[1].astype(jnp.int32)
    w = edge_weight_distance.astype(jnp.float32)
    if padn:
        spread = jnp.arange(padn, dtype=jnp.int32) % N
        row = jnp.concatenate([row, spread])
        col = jnp.concatenate([col, spread])
        w = jnp.concatenate([w, jnp.zeros((padn,), jnp.float32)])

    norm = _make_norm_kernel(N, EPAD)(row, col, w)

    pk = (row << 14) | col
    xflat = x.reshape(B * N * T, D)
    prop1 = _make_prop_kernel(
        N, D, M, EPAD, B * N * T, T,
        lambda m: (m // T) * (N * T) + (m % T))
    tx1 = prop1(xflat, pk, norm)

    prop2 = _make_prop_kernel(
        N, D, M, EPAD, M * N, 1,
        lambda m: m * N)
    p2 = prop2(tx1.reshape(M * N, D), pk, norm)

    PB = 1000
    NBLK = N // PB
    E2p = jnp.zeros((N, RP), jnp.float32).at[:, :R].set(E2.T)
    E1p = jnp.zeros((N, RP), jnp.float32).at[:, :R].set(E1)

    xv = x.reshape(B, N, T * D)
    tmp = pl.pallas_call(
        _tmp_body,
        grid=(M, NBLK),
        in_specs=[
            pl.BlockSpec((1, PB, D), lambda m, nb: (m // T, nb, m % T)),
            pl.BlockSpec((PB, RP), lambda m, nb: (nb, 0)),
        ],
        out_specs=pl.BlockSpec((1, RP, D), lambda m, nb: (m, 0, 0)),
        out_shape=jax.ShapeDtypeStruct((M, RP, D), jnp.float32),
    )(xv, E2p)

    full2 = lambda a, b: pl.BlockSpec((a, b), lambda m, nb: (0, 0))
    out = pl.pallas_call(
        _fused_body,
        grid=(M, NBLK),
        in_specs=[
            pl.BlockSpec((1, PB, D), lambda m, nb: (m // T, nb, m % T)),
            pl.BlockSpec((1, PB, D), lambda m, nb: (m, nb, 0)),
            pl.BlockSpec((1, PB, D), lambda m, nb: (m, nb, 0)),
            pl.BlockSpec((1, RP, D), lambda m, nb: (m, 0, 0)),
            pl.BlockSpec((PB, RP), lambda m, nb: (nb, 0)),
            full2(D, H),
            full2(1, H),
            pl.BlockSpec((3, D, H), lambda m, nb: (0, 0, 0)),
            full2(1, H),
            full2(H, H),
            full2(1, H),
            full2(H, H),
            full2(1, H),
        ],
        out_specs=pl.BlockSpec((1, PB, H), lambda m, nb: (m // T, nb, m % T)),
        out_shape=jax.ShapeDtypeStruct((B, N, T * H), jnp.float32),
    )(xv, tx1, p2, tmp, E1p,
      W_agcn, b_agcn.reshape(1, H), W_cheb, b_cheb.reshape(1, H),
      Wf_a, bf_a.reshape(1, H), Wf_g, bf_g.reshape(1, H))
    return out.reshape(B, N, T, H)


# speedup vs baseline: 13.9412x; 1.0090x over previous
import functools

import jax
import jax.numpy as jnp
from jax import lax
from jax.experimental import pallas as pl
from jax.experimental.pallas import tpu as pltpu
from jax.experimental.pallas import tpu_sc as plsc

NC = 2
NS = 16
LN = 16
CH = 128


def _rsqrt_newton(d):
    i = lax.bitcast_convert_type(d, jnp.int32)
    i = jnp.int32(0x5F3759DF) - (i >> 1)
    y = lax.bitcast_convert_type(i, jnp.float32)
    for _ in range(3):
        y = y * (1.5 - 0.5 * d * y * y)
    return y


def _make_norm_kernel(N, EPAD):
    NP = ((N + (LN * NS) - 1) // (LN * NS)) * (LN * NS)
    NPT = NP // NS
    EPT = EPAD // NS
    EPT2 = EPAD // (NS * NC)
    NCH = EPT // CH
    NCH2 = EPT2 // CH
    mesh = plsc.VectorSubcoreMesh(core_axis_name="c", subcore_axis_name="s")

    def body(row_hbm, col_hbm, w_hbm, norm_hbm,
             deg_sh, dinv_sh,
             zbuf, ist, wst, nodebuf, dloc, nbuf, sem):
        cid = lax.axis_index("c")
        sid = lax.axis_index("s")
        my0 = sid * NPT

        for i in range(NPT // LN):
            zbuf[pl.ds(i * LN, LN)] = jnp.zeros((LN,), jnp.float32)
        pltpu.sync_copy(zbuf, deg_sh.at[pl.ds(my0, NPT)])
        plsc.subcore_barrier()

        @pl.loop(0, NCH)
        def _deg(c):
            base = sid * EPT + c * CH
            pltpu.sync_copy(row_hbm.at[pl.ds(base, CH)], ist.at[0])
            pltpu.sync_copy(w_hbm.at[pl.ds(base, CH)], wst)
            pltpu.sync_copy(wst, deg_sh.at[ist.at[0]], add=True)

        plsc.subcore_barrier()

        pltpu.sync_copy(deg_sh.at[pl.ds(my0, NPT)], nodebuf)
        for i in range(NPT // LN):
            d = nodebuf[pl.ds(i * LN, LN)]
            y = _rsqrt_newton(d)
            nodebuf[pl.ds(i * LN, LN)] = jnp.where(d > 0.0, y, 0.0)
        pltpu.sync_copy(nodebuf, dinv_sh.at[pl.ds(my0, NPT)])
        plsc.subcore_barrier()

        pltpu.sync_copy(dinv_sh, dloc)

        @pl.loop(0, NCH2)
        def _norm(c):
            base = cid * (EPAD // NC) + sid * EPT2 + c * CH
            pltpu.sync_copy(row_hbm.at[pl.ds(base, CH)], ist.at[0])
            pltpu.sync_copy(col_hbm.at[pl.ds(base, CH)], ist.at[1])
            pltpu.sync_copy(w_hbm.at[pl.ds(base, CH)], wst)
            for g in range(CH // LN):
                rv = ist[0, pl.ds(g * LN, LN)]
                cv = ist[1, pl.ds(g * LN, LN)]
                wv = wst[pl.ds(g * LN, LN)]
                dr = plsc.load_gather(dloc, [rv])
                dc = plsc.load_gather(dloc, [cv])
                nbuf[pl.ds(g * LN, LN)] = -(wv * dr) * dc
            pltpu.sync_copy(nbuf, norm_hbm.at[pl.ds(base, CH)])

    kern = pl.kernel(
        body,
        out_type=jax.ShapeDtypeStruct((EPAD,), jnp.float32),
        mesh=mesh,
        compiler_params=pltpu.CompilerParams(needs_layout_passes=False),
        scratch_types=[
            pltpu.VMEM_SHARED((NP,), jnp.float32),
            pltpu.VMEM_SHARED((NP,), jnp.float32),
            pltpu.VMEM((NPT,), jnp.float32),
            pltpu.VMEM((2, CH), jnp.int32),
            pltpu.VMEM((CH,), jnp.float32),
            pltpu.VMEM((NPT,), jnp.float32),
            pltpu.VMEM((NP,), jnp.float32),
            pltpu.VMEM((CH,), jnp.float32),
            pltpu.SemaphoreType.DMA,
        ],
    )
    return kern


def _make_prop_kernel(N, D, M, EPAD, src_rows, stride, base_of_m):
    NPT = ((N // NS + 7) // 8) * 8
    NP = NPT * NS
    EPT = EPAD // NS
    NCH = EPT // CH
    MC = M // NC
    ZR = 64
    mesh = plsc.VectorSubcoreMesh(core_axis_name="c", subcore_axis_name="s")

    def body(src_hbm, pk_hbm, norm_hbm, dst_hbm,
             acc_sh, zbuf, pkst, nst, gidx, scidx,
             pay0, pay1, si0, si1, sp0, sp1, ss0, ss1, sz, so):
        cid = lax.axis_index("c")
        sid = lax.axis_index("s")
        my0 = sid * NPT
        pays = (pay0, pay1)
        sis = (si0, si1)
        sps = (sp0, sp1)
        sss = (ss0, ss1)

        for i in range(ZR * D // LN):
            zbuf[(i * LN) // D, pl.ds((i * LN) % D, LN)] = jnp.zeros((LN,), jnp.float32)

        def idx_start(b, c):
            base = sid * EPT + c * CH
            pltpu.async_copy(pk_hbm.at[pl.ds(base, CH)], pkst.at[b], sis[b])
            pltpu.async_copy(norm_hbm.at[pl.ds(base, CH)], nst.at[b], sis[b])

        def idx_wait(b):
            pltpu.make_async_copy(pk_hbm.at[pl.ds(0, CH)], pkst.at[b], sis[b]).wait()
            pltpu.make_async_copy(norm_hbm.at[pl.ds(0, CH)], nst.at[b], sis[b]).wait()

        def gs_compute(b, base_m):
            for g in range(CH // LN):
                pv = pkst[b, pl.ds(g * LN, LN)]
                gidx[b, pl.ds(g * LN, LN)] = (pv >> 14) * stride + base_m
                scidx[b, pl.ds(g * LN, LN)] = pv & 16383

        def gather_start(b):
            pltpu.async_copy(src_hbm.at[gidx.at[b]], pays[b], sps[b])

        def gather_wait(b):
            pltpu.make_async_copy(src_hbm.at[gidx.at[b]], pays[b], sps[b]).wait()

        def scale(b):
            pay = pays[b]
            @pl.loop(0, CH // LN)
            def _grp(g):
                nv = nst[b, pl.ds(g * LN, LN)]
                for i in range(LN):
                    for j in range(D // LN):
                        v = pay[g * LN + i, pl.ds(j * LN, LN)]
                        pay[g * LN + i, pl.ds(j * LN, LN)] = v * nv[i]

        def scatter_start(b):
            pltpu.async_copy(pays[b], acc_sh.at[scidx.at[b]], sss[b], add=True)

        def scatter_wait(b):
            pltpu.make_async_copy(pays[b], acc_sh.at[scidx.at[b]], sss[b]).wait()

        nfull = N // NPT
        rem = N - nfull * NPT

        def copyout_start(m):
            if rem == 0:
                pltpu.async_copy(acc_sh.at[pl.ds(my0, NPT)],
                                 dst_hbm.at[m].at[pl.ds(my0, NPT)], so)
            else:
                @pl.when(sid < nfull)
                def _full():
                    pltpu.async_copy(acc_sh.at[pl.ds(my0, NPT)],
                                     dst_hbm.at[m].at[pl.ds(my0, NPT)], so)

                @pl.when(sid == nfull)
                def _tail():
                    pltpu.async_copy(acc_sh.at[pl.ds(my0, rem)],
                                     dst_hbm.at[m].at[pl.ds(my0, rem)], so)

        def copyout_wait():
            if rem == 0:
                pltpu.make_async_copy(acc_sh.at[pl.ds(my0, NPT)],
                                      dst_hbm.at[0].at[pl.ds(my0, NPT)], so).wait()
            else:
                @pl.when(sid < nfull)
                def _full():
                    pltpu.make_async_copy(acc_sh.at[pl.ds(my0, NPT)],
                                          dst_hbm.at[0].at[pl.ds(my0, NPT)], so).wait()

                @pl.when(sid == nfull)
                def _tail():
                    pltpu.make_async_copy(acc_sh.at[pl.ds(my0, rem)],
                                          dst_hbm.at[0].at[pl.ds(my0, rem)], so).wait()

        @pl.loop(0, MC)
        def _inst(inst):
            m = inst * NC + cid
            base_m = base_of_m(m)

            idx_start(0, 0)
            idx_start(1, 1)

            @pl.when(inst > 0)
            def _():
                copyout_wait()

            for z in range(NPT // ZR):
                pltpu.async_copy(zbuf, acc_sh.at[pl.ds(my0 + z * ZR, ZR)], sz)
            if NPT % ZR:
                pltpu.async_copy(zbuf.at[pl.ds(0, NPT % ZR)],
                                 acc_sh.at[pl.ds(my0 + (NPT // ZR) * ZR, NPT % ZR)], sz)

            idx_wait(0)
            gs_compute(0, base_m)
            gather_start(0)
            idx_wait(1)
            gs_compute(1, base_m)
            gather_start(1)

            for z in range(NPT // ZR):
                pltpu.make_async_copy(zbuf, acc_sh.at[pl.ds(my0, ZR)], sz).wait()
            if NPT % ZR:
                pltpu.make_async_copy(zbuf.at[pl.ds(0, NPT % ZR)],
                                      acc_sh.at[pl.ds(my0, NPT % ZR)], sz).wait()
            plsc.subcore_barrier()

            gather_wait(0)
            scale(0)
            idx_start(0, 2)
            scatter_start(0)

            @pl.loop(0, (NCH - 2) // 2)
            def _pair(p):
                for b, cc in ((1, 2 * p + 1), (0, 2 * p + 2)):
                    nb = 1 - b
                    idx_wait(nb)
                    scatter_wait(nb)
                    gs_compute(nb, base_m)
                    gather_start(nb)
                    gather_wait(b)
                    scale(b)
                    if b == 1:
                        idx_start(b, 2 * p + 3)
                    else:
                        @pl.when(2 * p + 4 < NCH)
                        def _():
                            idx_start(0, 2 * p + 4)
                    scatter_start(b)

            gather_wait(1)
            scale(1)
            scatter_start(1)
            scatter_wait(0)
            scatter_wait(1)

            plsc.subcore_barrier()

            copyout_start(m)

        copyout_wait()

    kern = pl.kernel(
        body,
        out_type=jax.ShapeDtypeStruct((M, N, D), jnp.float32),
        mesh=mesh,
        compiler_params=pltpu.CompilerParams(needs_layout_passes=False),
        scratch_types=[
            pltpu.VMEM_SHARED((NP, D), jnp.float32),
            pltpu.VMEM((ZR, D), jnp.float32),
            pltpu.VMEM((2, CH), jnp.int32),
            pltpu.VMEM((2, CH), jnp.float32),
            pltpu.VMEM((2, CH), jnp.int32),
            pltpu.VMEM((2, CH), jnp.int32),
            pltpu.VMEM((CH, D), jnp.float32),
            pltpu.VMEM((CH, D), jnp.float32),
            pltpu.SemaphoreType.DMA,
            pltpu.SemaphoreType.DMA,
            pltpu.SemaphoreType.DMA,
            pltpu.SemaphoreType.DMA,
            pltpu.SemaphoreType.DMA,
            pltpu.SemaphoreType.DMA,
            pltpu.SemaphoreType.DMA,
            pltpu.SemaphoreType.DMA,
        ],
    )
    return kern


def _tmp_body(x_ref, e2_ref, out_ref):
    nb = pl.program_id(1)

    @pl.when(nb == 0)
    def _():
        out_ref[...] = jnp.zeros_like(out_ref)

    xb = x_ref[0]
    e2b = e2_ref[...]
    out_ref[0] += lax.dot_general(e2b, xb, (((0,), (0,)), ((), ())),
                                  preferred_element_type=jnp.float32)


def _fused_body(x_ref, t1_ref, p2_ref, tmp_ref, e1_ref,
                wa_ref, ba_ref, wc_ref, bc_ref, wfa_ref, bfa_ref,
                wfg_ref, bfg_ref, out_ref):
    xb = x_ref[0]
    t1b = t1_ref[0]
    p2b = p2_ref[0]
    tmpm = tmp_ref[0]
    e1b = e1_ref[...]
    dot = functools.partial(jnp.dot, preferred_element_type=jnp.float32)

    h = jax.nn.relu(dot(e1b, tmpm))
    agcn = dot(h, wa_ref[...]) + ba_ref[0]
    w0 = wc_ref[0]
    w1 = wc_ref[1]
    w2 = wc_ref[2]
    cheb = (dot(xb, w0 - w2) + dot(t1b, w1) + 2.0 * dot(p2b, w2) + bc_ref[0])
    gate = jax.nn.sigmoid(dot(agcn, wfa_ref[...]) + bfa_ref[0]
                          + dot(cheb, wfg_ref[...]) + bfg_ref[0])
    out_ref[0] = gate * agcn + (1.0 - gate) * cheb


def kernel(x, edge_index_distance, edge_weight_distance, E1, E2,
           W_agcn, b_agcn, W_cheb, b_cheb, Wf_a, bf_a, Wf_g, bf_g):
    B, N, T, D = x.shape
    M = B * T
    H = W_agcn.shape[1]
    R = E1.shape[1]
    E = edge_weight_distance.shape[0]
    RP = ((R + 7) // 8) * 8

    EUNIT = NS * CH * 8
    EPAD = ((E + EUNIT - 1) // EUNIT) * EUNIT
    padn = EPAD - E
    row = edge_index_distance[0].astype(jnp.int32)
    col = edge_index_distance[1].astype(jnp.int32)
    w = edge_weight_distance.astype(jnp.float32)
    if padn:
        spread = jnp.arange(padn, dtype=jnp.int32) % N
        row = jnp.concatenate([row, spread])
        col = jnp.concatenate([col, spread])
        w = jnp.concatenate([w, jnp.zeros((padn,), jnp.float32)])

    norm = _make_norm_kernel(N, EPAD)(row, col, w)

    pk = (row << 14) | col
    xflat = x.reshape(B * N * T, D)
    prop1 = _make_prop_kernel(
        N, D, M, EPAD, B * N * T, T,
        lambda m: (m // T) * (N * T) + (m % T))
    tx1 = prop1(xflat, pk, norm)

    prop2 = _make_prop_kernel(
        N, D, M, EPAD, M * N, 1,
        lambda m: m * N)
    p2 = prop2(tx1.reshape(M * N, D), pk, norm)

    PB = 1000
    NBLK = N // PB
    E2p = jnp.zeros((N, RP), jnp.float32).at[:, :R].set(E2.T)
    E1p = jnp.zeros((N, RP), jnp.float32).at[:, :R].set(E1)

    xv = x.reshape(B, N, T * D)
    tmp = pl.pallas_call(
        _tmp_body,
        grid=(M, NBLK),
        in_specs=[
            pl.BlockSpec((1, PB, D), lambda m, nb: (m // T, nb, m % T)),
            pl.BlockSpec((PB, RP), lambda m, nb: (nb, 0)),
        ],
        out_specs=pl.BlockSpec((1, RP, D), lambda m, nb: (m, 0, 0)),
        out_shape=jax.ShapeDtypeStruct((M, RP, D), jnp.float32),
    )(xv, E2p)

    full2 = lambda a, b: pl.BlockSpec((a, b), lambda m, nb: (0, 0))
    out = pl.pallas_call(
        _fused_body,
        grid=(M, NBLK),
        in_specs=[
            pl.BlockSpec((1, PB, D), lambda m, nb: (m // T, nb, m % T)),
            pl.BlockSpec((1, PB, D), lambda m, nb: (m, nb, 0)),
            pl.BlockSpec((1, PB, D), lambda m, nb: (m, nb, 0)),
            pl.BlockSpec((1, RP, D), lambda m, nb: (m, 0, 0)),
            pl.BlockSpec((PB, RP), lambda m, nb: (nb, 0)),
            full2(D, H),
            full2(1, H),
            pl.BlockSpec((3, D, H), lambda m, nb: (0, 0, 0)),
            full2(1, H),
            full2(H, H),
            full2(1, H),
            full2(H, H),
            full2(1, H),
        ],
        out_specs=pl.BlockSpec((1, PB, H), lambda m, nb: (m // T, nb, m % T)),
        out_shape=jax.ShapeDtypeStruct((B, N, T * H), jnp.float32),
    )(xv, tx1, p2, tmp, E1p,
      W_agcn, b_agcn.reshape(1, H), W_cheb, b_cheb.reshape(1, H),
      Wf_a, bf_a.reshape(1, H), Wf_g, bf_g.reshape(1, H))
    return out.reshape(B, N, T, H)


# speedup vs baseline: 14.6570x; 1.0513x over previous
import functools

import jax
import jax.numpy as jnp
from jax import lax
from jax.experimental import pallas as pl
from jax.experimental.pallas import tpu as pltpu
from jax.experimental.pallas import tpu_sc as plsc

NC = 2
NS = 16
LN = 16
CH = 128


def _rsqrt_newton(d):
    i = lax.bitcast_convert_type(d, jnp.int32)
    i = jnp.int32(0x5F3759DF) - (i >> 1)
    y = lax.bitcast_convert_type(i, jnp.float32)
    for _ in range(3):
        y = y * (1.5 - 0.5 * d * y * y)
    return y


def _make_norm_kernel(N, EPAD):
    NP = ((N + (LN * NS) - 1) // (LN * NS)) * (LN * NS)
    NPT = NP // NS
    EPT = EPAD // NS
    EPT2 = EPAD // (NS * NC)
    NCH = EPT // CH
    NCH2 = EPT2 // CH
    mesh = plsc.VectorSubcoreMesh(core_axis_name="c", subcore_axis_name="s")

    def body(pk_hbm, w_hbm, norm_hbm,
             deg_sh, dinv_sh,
             zbuf, pkst, wstb, ridx, nodebuf, dloc, nbuf,
             sl0, sl1, sw0, sw1):
        cid = lax.axis_index("c")
        sid = lax.axis_index("s")
        my0 = sid * NPT
        sls = (sl0, sl1)
        sws = (sw0, sw1)

        def load(b, base):
            pltpu.async_copy(pk_hbm.at[pl.ds(base, CH)], pkst.at[b], sls[b])
            pltpu.async_copy(w_hbm.at[pl.ds(base, CH)], wstb.at[b], sls[b])

        def load_wait(b):
            pltpu.make_async_copy(pk_hbm.at[pl.ds(0, CH)], pkst.at[b], sls[b]).wait()
            pltpu.make_async_copy(w_hbm.at[pl.ds(0, CH)], wstb.at[b], sls[b]).wait()

        for i in range(NPT // LN):
            zbuf[pl.ds(i * LN, LN)] = jnp.zeros((LN,), jnp.float32)
        pltpu.sync_copy(zbuf, deg_sh.at[pl.ds(my0, NPT)])
        plsc.subcore_barrier()

        dbase = lambda c: sid * EPT + c * CH
        load(0, dbase(0))
        load(1, dbase(1))

        @pl.loop(0, NCH // 2)
        def _deg(p):
            for b in (0, 1):
                cc = 2 * p + b
                load_wait(b)
                for g in range(CH // LN):
                    ridx[b, pl.ds(g * LN, LN)] = pkst[b, pl.ds(g * LN, LN)] >> 14
                pltpu.sync_copy(wstb.at[b], deg_sh.at[ridx.at[b]], add=True)

                @pl.when(cc + 2 < NCH)
                def _():
                    load(b, dbase(cc + 2))

        plsc.subcore_barrier()

        pltpu.sync_copy(deg_sh.at[pl.ds(my0, NPT)], nodebuf)
        for i in range(NPT // LN):
            d = nodebuf[pl.ds(i * LN, LN)]
            y = _rsqrt_newton(d)
            nodebuf[pl.ds(i * LN, LN)] = jnp.where(d > 0.0, y, 0.0)
        pltpu.sync_copy(nodebuf, dinv_sh.at[pl.ds(my0, NPT)])
        plsc.subcore_barrier()

        pltpu.sync_copy(dinv_sh, dloc)

        nbase = lambda c: cid * (EPAD // NC) + sid * EPT2 + c * CH
        load(0, nbase(0))
        load(1, nbase(1))

        @pl.loop(0, NCH2 // 2)
        def _norm(p):
            for b in (0, 1):
                cc = 2 * p + b
                load_wait(b)

                @pl.when(cc >= 2)
                def _():
                    pltpu.make_async_copy(nbuf.at[b], norm_hbm.at[pl.ds(0, CH)],
                                          sws[b]).wait()

                for g in range(CH // LN):
                    pv = pkst[b, pl.ds(g * LN, LN)]
                    wv = wstb[b, pl.ds(g * LN, LN)]
                    dr = plsc.load_gather(dloc, [pv >> 14])
                    dc = plsc.load_gather(dloc, [pv & 16383])
                    nbuf[b, pl.ds(g * LN, LN)] = -(wv * dr) * dc
                pltpu.async_copy(nbuf.at[b], norm_hbm.at[pl.ds(nbase(cc), CH)],
                                 sws[b])

                @pl.when(cc + 2 < NCH2)
                def _():
                    load(b, nbase(cc + 2))

        for b in (0, 1):
            pltpu.make_async_copy(nbuf.at[b], norm_hbm.at[pl.ds(0, CH)],
                                  sws[b]).wait()

    kern = pl.kernel(
        body,
        out_type=jax.ShapeDtypeStruct((EPAD,), jnp.float32),
        mesh=mesh,
        compiler_params=pltpu.CompilerParams(needs_layout_passes=False),
        scratch_types=[
            pltpu.VMEM_SHARED((NP,), jnp.float32),
            pltpu.VMEM_SHARED((NP,), jnp.float32),
            pltpu.VMEM((NPT,), jnp.float32),
            pltpu.VMEM((2, CH), jnp.int32),
            pltpu.VMEM((2, CH), jnp.float32),
            pltpu.VMEM((2, CH), jnp.int32),
            pltpu.VMEM((NPT,), jnp.float32),
            pltpu.VMEM((NP,), jnp.float32),
            pltpu.VMEM((2, CH), jnp.float32),
            pltpu.SemaphoreType.DMA,
            pltpu.SemaphoreType.DMA,
            pltpu.SemaphoreType.DMA,
            pltpu.SemaphoreType.DMA,
        ],
    )
    return kern


def _make_prop_kernel(N, D, M, EPAD, src_rows, stride, base_of_m):
    NPT = ((N // NS + 7) // 8) * 8
    NP = NPT * NS
    EPT = EPAD // NS
    NCH = EPT // CH
    MC = M // NC
    ZR = 64
    mesh = plsc.VectorSubcoreMesh(core_axis_name="c", subcore_axis_name="s")

    def body(src_hbm, pk_hbm, norm_hbm, dst_hbm,
             acc_sh, zbuf, pkst, nst, gidx, scidx,
             pay0, pay1, si0, si1, sp0, sp1, ss0, ss1, sz, so):
        cid = lax.axis_index("c")
        sid = lax.axis_index("s")
        my0 = sid * NPT
        pays = (pay0, pay1)
        sis = (si0, si1)
        sps = (sp0, sp1)
        sss = (ss0, ss1)

        for i in range(ZR * D // LN):
            zbuf[(i * LN) // D, pl.ds((i * LN) % D, LN)] = jnp.zeros((LN,), jnp.float32)

        def idx_start(b, c):
            base = sid * EPT + c * CH
            pltpu.async_copy(pk_hbm.at[pl.ds(base, CH)], pkst.at[b], sis[b])
            pltpu.async_copy(norm_hbm.at[pl.ds(base, CH)], nst.at[b], sis[b])

        def idx_wait(b):
            pltpu.make_async_copy(pk_hbm.at[pl.ds(0, CH)], pkst.at[b], sis[b]).wait()
            pltpu.make_async_copy(norm_hbm.at[pl.ds(0, CH)], nst.at[b], sis[b]).wait()

        def gs_compute(b, base_m):
            for g in range(CH // LN):
                pv = pkst[b, pl.ds(g * LN, LN)]
                gidx[b, pl.ds(g * LN, LN)] = (pv >> 14) * stride + base_m
                scidx[b, pl.ds(g * LN, LN)] = pv & 16383

        def gather_start(b):
            pltpu.async_copy(src_hbm.at[gidx.at[b]], pays[b], sps[b])

        def gather_wait(b):
            pltpu.make_async_copy(src_hbm.at[gidx.at[b]], pays[b], sps[b]).wait()

        def scale(b):
            pay = pays[b]
            @pl.loop(0, CH // LN)
            def _grp(g):
                nv = nst[b, pl.ds(g * LN, LN)]
                for i in range(LN):
                    for j in range(D // LN):
                        v = pay[g * LN + i, pl.ds(j * LN, LN)]
                        pay[g * LN + i, pl.ds(j * LN, LN)] = v * nv[i]

        def scatter_start(b):
            pltpu.async_copy(pays[b], acc_sh.at[scidx.at[b]], sss[b], add=True)

        def scatter_wait(b):
            pltpu.make_async_copy(pays[b], acc_sh.at[scidx.at[b]], sss[b]).wait()

        nfull = N // NPT
        rem = N - nfull * NPT

        def copyout_start(m):
            if rem == 0:
                pltpu.async_copy(acc_sh.at[pl.ds(my0, NPT)],
                                 dst_hbm.at[m].at[pl.ds(my0, NPT)], so)
            else:
                @pl.when(sid < nfull)
                def _full():
                    pltpu.async_copy(acc_sh.at[pl.ds(my0, NPT)],
                                     dst_hbm.at[m].at[pl.ds(my0, NPT)], so)

                @pl.when(sid == nfull)
                def _tail():
                    pltpu.async_copy(acc_sh.at[pl.ds(my0, rem)],
                                     dst_hbm.at[m].at[pl.ds(my0, rem)], so)

        def copyout_wait():
            if rem == 0:
                pltpu.make_async_copy(acc_sh.at[pl.ds(my0, NPT)],
                                      dst_hbm.at[0].at[pl.ds(my0, NPT)], so).wait()
            else:
                @pl.when(sid < nfull)
                def _full():
                    pltpu.make_async_copy(acc_sh.at[pl.ds(my0, NPT)],
                                          dst_hbm.at[0].at[pl.ds(my0, NPT)], so).wait()

                @pl.when(sid == nfull)
                def _tail():
                    pltpu.make_async_copy(acc_sh.at[pl.ds(my0, rem)],
                                          dst_hbm.at[0].at[pl.ds(my0, rem)], so).wait()

        @pl.loop(0, MC)
        def _inst(inst):
            m = inst * NC + cid
            base_m = base_of_m(m)

            idx_start(0, 0)
            idx_start(1, 1)

            @pl.when(inst > 0)
            def _():
                copyout_wait()

            for z in range(NPT // ZR):
                pltpu.async_copy(zbuf, acc_sh.at[pl.ds(my0 + z * ZR, ZR)], sz)
            if NPT % ZR:
                pltpu.async_copy(zbuf.at[pl.ds(0, NPT % ZR)],
                                 acc_sh.at[pl.ds(my0 + (NPT // ZR) * ZR, NPT % ZR)], sz)

            idx_wait(0)
            gs_compute(0, base_m)
            gather_start(0)
            idx_wait(1)
            gs_compute(1, base_m)
            gather_start(1)

            for z in range(NPT // ZR):
                pltpu.make_async_copy(zbuf, acc_sh.at[pl.ds(my0, ZR)], sz).wait()
            if NPT % ZR:
                pltpu.make_async_copy(zbuf.at[pl.ds(0, NPT % ZR)],
                                      acc_sh.at[pl.ds(my0, NPT % ZR)], sz).wait()
            plsc.subcore_barrier()

            gather_wait(0)
            scale(0)
            idx_start(0, 2)
            scatter_start(0)

            @pl.loop(0, (NCH - 2) // 2)
            def _pair(p):
                for b, cc in ((1, 2 * p + 1), (0, 2 * p + 2)):
                    nb = 1 - b
                    idx_wait(nb)
                    scatter_wait(nb)
                    gs_compute(nb, base_m)
                    gather_start(nb)
                    gather_wait(b)
                    scale(b)
                    if b == 1:
                        idx_start(b, 2 * p + 3)
                    else:
                        @pl.when(2 * p + 4 < NCH)
                        def _():
                            idx_start(0, 2 * p + 4)
                    scatter_start(b)

            gather_wait(1)
            scale(1)
            scatter_start(1)
            scatter_wait(0)
            scatter_wait(1)

            plsc.subcore_barrier()

            copyout_start(m)

        copyout_wait()

    kern = pl.kernel(
        body,
        out_type=jax.ShapeDtypeStruct((M, N, D), jnp.float32),
        mesh=mesh,
        compiler_params=pltpu.CompilerParams(needs_layout_passes=False),
        scratch_types=[
            pltpu.VMEM_SHARED((NP, D), jnp.float32),
            pltpu.VMEM((ZR, D), jnp.float32),
            pltpu.VMEM((2, CH), jnp.int32),
            pltpu.VMEM((2, CH), jnp.float32),
            pltpu.VMEM((2, CH), jnp.int32),
            pltpu.VMEM((2, CH), jnp.int32),
            pltpu.VMEM((CH, D), jnp.float32),
            pltpu.VMEM((CH, D), jnp.float32),
            pltpu.SemaphoreType.DMA,
            pltpu.SemaphoreType.DMA,
            pltpu.SemaphoreType.DMA,
            pltpu.SemaphoreType.DMA,
            pltpu.SemaphoreType.DMA,
            pltpu.SemaphoreType.DMA,
            pltpu.SemaphoreType.DMA,
            pltpu.SemaphoreType.DMA,
        ],
    )
    return kern


def _tmp_body(x_ref, e2_ref, out_ref):
    nb = pl.program_id(1)

    @pl.when(nb == 0)
    def _():
        out_ref[...] = jnp.zeros_like(out_ref)

    xb = x_ref[0]
    e2b = e2_ref[...]
    out_ref[0] += lax.dot_general(e2b, xb, (((0,), (0,)), ((), ())),
                                  preferred_element_type=jnp.float32)


def _fused_body(x_ref, t1_ref, p2_ref, tmp_ref, e1_ref,
                wa_ref, ba_ref, wc_ref, bc_ref, wfa_ref, bfa_ref,
                wfg_ref, bfg_ref, out_ref):
    xb = x_ref[0]
    t1b = t1_ref[0]
    p2b = p2_ref[0]
    tmpm = tmp_ref[0]
    e1b = e1_ref[...]
    dot = functools.partial(jnp.dot, preferred_element_type=jnp.float32)

    h = jax.nn.relu(dot(e1b, tmpm))
    agcn = dot(h, wa_ref[...]) + ba_ref[0]
    w0 = wc_ref[0]
    w1 = wc_ref[1]
    w2 = wc_ref[2]
    cheb = (dot(xb, w0 - w2) + dot(t1b, w1) + 2.0 * dot(p2b, w2) + bc_ref[0])
    gate = jax.nn.sigmoid(dot(agcn, wfa_ref[...]) + bfa_ref[0]
                          + dot(cheb, wfg_ref[...]) + bfg_ref[0])
    out_ref[0] = gate * agcn + (1.0 - gate) * cheb


def kernel(x, edge_index_distance, edge_weight_distance, E1, E2,
           W_agcn, b_agcn, W_cheb, b_cheb, Wf_a, bf_a, Wf_g, bf_g):
    B, N, T, D = x.shape
    M = B * T
    H = W_agcn.shape[1]
    R = E1.shape[1]
    E = edge_weight_distance.shape[0]
    RP = ((R + 7) // 8) * 8

    EUNIT = NS * CH * 8
    EPAD = ((E + EUNIT - 1) // EUNIT) * EUNIT
    padn = EPAD - E
    row = edge_index_distance[0].astype(jnp.int32)
    col = edge_index_distance[1].astype(jnp.int32)
    w = edge_weight_distance.astype(jnp.float32)
    if padn:
        spread = jnp.arange(padn, dtype=jnp.int32) % N
        row = jnp.concatenate([row, spread])
        col = jnp.concatenate([col, spread])
        w = jnp.concatenate([w, jnp.zeros((padn,), jnp.float32)])

    pk = (row << 14) | col
    norm = _make_norm_kernel(N, EPAD)(pk, w)
    xflat = x.reshape(B * N * T, D)
    prop1 = _make_prop_kernel(
        N, D, M, EPAD, B * N * T, T,
        lambda m: (m // T) * (N * T) + (m % T))
    tx1 = prop1(xflat, pk, norm)

    prop2 = _make_prop_kernel(
        N, D, M, EPAD, M * N, 1,
        lambda m: m * N)
    p2 = prop2(tx1.reshape(M * N, D), pk, norm)

    PB = 1000
    NBLK = N // PB
    E2p = jnp.zeros((N, RP), jnp.float32).at[:, :R].set(E2.T)
    E1p = jnp.zeros((N, RP), jnp.float32).at[:, :R].set(E1)

    xv = x.reshape(B, N, T * D)
    tmp = pl.pallas_call(
        _tmp_body,
        grid=(M, NBLK),
        in_specs=[
            pl.BlockSpec((1, PB, D), lambda m, nb: (m // T, nb, m % T)),
            pl.BlockSpec((PB, RP), lambda m, nb: (nb, 0)),
        ],
        out_specs=pl.BlockSpec((1, RP, D), lambda m, nb: (m, 0, 0)),
        out_shape=jax.ShapeDtypeStruct((M, RP, D), jnp.float32),
    )(xv, E2p)

    full2 = lambda a, b: pl.BlockSpec((a, b), lambda m, nb: (0, 0))
    out = pl.pallas_call(
        _fused_body,
        grid=(M, NBLK),
        in_specs=[
            pl.BlockSpec((1, PB, D), lambda m, nb: (m // T, nb, m % T)),
            pl.BlockSpec((1, PB, D), lambda m, nb: (m, nb, 0)),
            pl.BlockSpec((1, PB, D), lambda m, nb: (m, nb, 0)),
            pl.BlockSpec((1, RP, D), lambda m, nb: (m, 0, 0)),
            pl.BlockSpec((PB, RP), lambda m, nb: (nb, 0)),
            full2(D, H),
            full2(1, H),
            pl.BlockSpec((3, D, H), lambda m, nb: (0, 0, 0)),
            full2(1, H),
            full2(H, H),
            full2(1, H),
            full2(H, H),
            full2(1, H),
        ],
        out_specs=pl.BlockSpec((1, PB, H), lambda m, nb: (m // T, nb, m % T)),
        out_shape=jax.ShapeDtypeStruct((B, N, T * H), jnp.float32),
    )(xv, tx1, p2, tmp, E1p,
      W_agcn, b_agcn.reshape(1, H), W_cheb, b_cheb.reshape(1, H),
      Wf_a, bf_a.reshape(1, H), Wf_g, bf_g.reshape(1, H))
    return out.reshape(B, N, T, H)


# speedup vs baseline: 14.6571x; 1.0000x over previous
import functools

import jax
import jax.numpy as jnp
from jax import lax
from jax.experimental import pallas as pl
from jax.experimental.pallas import tpu as pltpu
from jax.experimental.pallas import tpu_sc as plsc

NC = 2
NS = 16
LN = 16
CH = 128


def _rsqrt_newton(d):
    i = lax.bitcast_convert_type(d, jnp.int32)
    i = jnp.int32(0x5F3759DF) - (i >> 1)
    y = lax.bitcast_convert_type(i, jnp.float32)
    for _ in range(3):
        y = y * (1.5 - 0.5 * d * y * y)
    return y


def _make_norm_kernel(N, EPAD):
    NP = ((N + (LN * NS) - 1) // (LN * NS)) * (LN * NS)
    NPT = NP // NS
    EPT = EPAD // NS
    EPT2 = EPAD // (NS * NC)
    NCH = EPT // CH
    NCH2 = EPT2 // CH
    mesh = plsc.VectorSubcoreMesh(core_axis_name="c", subcore_axis_name="s")

    def body(pk_hbm, w_hbm, norm_hbm,
             deg_sh, dinv_sh,
             zbuf, pkst, wstb, ridx, nodebuf, dloc, nbuf,
             sl0, sl1, sw0, sw1):
        cid = lax.axis_index("c")
        sid = lax.axis_index("s")
        my0 = sid * NPT
        sls = (sl0, sl1)
        sws = (sw0, sw1)

        def load(b, base):
            pltpu.async_copy(pk_hbm.at[pl.ds(base, CH)], pkst.at[b], sls[b])
            pltpu.async_copy(w_hbm.at[pl.ds(base, CH)], wstb.at[b], sls[b])

        def load_wait(b):
            pltpu.make_async_copy(pk_hbm.at[pl.ds(0, CH)], pkst.at[b], sls[b]).wait()
            pltpu.make_async_copy(w_hbm.at[pl.ds(0, CH)], wstb.at[b], sls[b]).wait()

        for i in range(NPT // LN):
            zbuf[pl.ds(i * LN, LN)] = jnp.zeros((LN,), jnp.float32)
        pltpu.sync_copy(zbuf, deg_sh.at[pl.ds(my0, NPT)])
        plsc.subcore_barrier()

        dbase = lambda c: sid * EPT + c * CH
        load(0, dbase(0))
        load(1, dbase(1))

        @pl.loop(0, NCH // 2)
        def _deg(p):
            for b in (0, 1):
                cc = 2 * p + b
                load_wait(b)
                for g in range(CH // LN):
                    ridx[b, pl.ds(g * LN, LN)] = pkst[b, pl.ds(g * LN, LN)] >> 14
                pltpu.sync_copy(wstb.at[b], deg_sh.at[ridx.at[b]], add=True)

                @pl.when(cc + 2 < NCH)
                def _():
                    load(b, dbase(cc + 2))

        plsc.subcore_barrier()

        pltpu.sync_copy(deg_sh.at[pl.ds(my0, NPT)], nodebuf)
        for i in range(NPT // LN):
            d = nodebuf[pl.ds(i * LN, LN)]
            y = _rsqrt_newton(d)
            nodebuf[pl.ds(i * LN, LN)] = jnp.where(d > 0.0, y, 0.0)
        pltpu.sync_copy(nodebuf, dinv_sh.at[pl.ds(my0, NPT)])
        plsc.subcore_barrier()

        pltpu.sync_copy(dinv_sh, dloc)

        nbase = lambda c: cid * (EPAD // NC) + sid * EPT2 + c * CH
        load(0, nbase(0))
        load(1, nbase(1))

        @pl.loop(0, NCH2 // 2)
        def _norm(p):
            for b in (0, 1):
                cc = 2 * p + b
                load_wait(b)

                @pl.when(cc >= 2)
                def _():
                    pltpu.make_async_copy(nbuf.at[b], norm_hbm.at[pl.ds(0, CH)],
                                          sws[b]).wait()

                for g in range(CH // LN):
                    pv = pkst[b, pl.ds(g * LN, LN)]
                    wv = wstb[b, pl.ds(g * LN, LN)]
                    dr = plsc.load_gather(dloc, [pv >> 14])
                    dc = plsc.load_gather(dloc, [pv & 16383])
                    nbuf[b, pl.ds(g * LN, LN)] = -(wv * dr) * dc
                pltpu.async_copy(nbuf.at[b], norm_hbm.at[pl.ds(nbase(cc), CH)],
                                 sws[b])

                @pl.when(cc + 2 < NCH2)
                def _():
                    load(b, nbase(cc + 2))

        for b in (0, 1):
            pltpu.make_async_copy(nbuf.at[b], norm_hbm.at[pl.ds(0, CH)],
                                  sws[b]).wait()

    kern = pl.kernel(
        body,
        out_type=jax.ShapeDtypeStruct((EPAD,), jnp.float32),
        mesh=mesh,
        compiler_params=pltpu.CompilerParams(needs_layout_passes=False),
        scratch_types=[
            pltpu.VMEM_SHARED((NP,), jnp.float32),
            pltpu.VMEM_SHARED((NP,), jnp.float32),
            pltpu.VMEM((NPT,), jnp.float32),
            pltpu.VMEM((2, CH), jnp.int32),
            pltpu.VMEM((2, CH), jnp.float32),
            pltpu.VMEM((2, CH), jnp.int32),
            pltpu.VMEM((NPT,), jnp.float32),
            pltpu.VMEM((NP,), jnp.float32),
            pltpu.VMEM((2, CH), jnp.float32),
            pltpu.SemaphoreType.DMA,
            pltpu.SemaphoreType.DMA,
            pltpu.SemaphoreType.DMA,
            pltpu.SemaphoreType.DMA,
        ],
    )
    return kern


def _make_prop_kernel(N, D, M, EPAD, src_rows, stride, base_of_m):
    NPT = ((N // NS + 7) // 8) * 8
    NP = NPT * NS
    EPT = EPAD // NS
    NCH = EPT // CH
    MC = M // NC
    ZR = 64
    mesh = plsc.VectorSubcoreMesh(core_axis_name="c", subcore_axis_name="s")

    def body(src_hbm, pk_hbm, norm_hbm, dst_hbm,
             acc_sh, zbuf, pkst, nst, gidx, scidx,
             pay0, pay1, si0, si1, sp0, sp1, ss0, ss1, sz, so):
        cid = lax.axis_index("c")
        sid = lax.axis_index("s")
        my0 = sid * NPT
        pays = (pay0, pay1)
        sis = (si0, si1)
        sps = (sp0, sp1)
        sss = (ss0, ss1)

        for i in range(ZR * D // LN):
            zbuf[(i * LN) // D, pl.ds((i * LN) % D, LN)] = jnp.zeros((LN,), jnp.float32)

        def idx_start(b, c):
            base = sid * EPT + c * CH
            pltpu.async_copy(pk_hbm.at[pl.ds(base, CH)], pkst.at[b], sis[b])
            pltpu.async_copy(norm_hbm.at[pl.ds(base, CH)], nst.at[b], sis[b])

        def idx_wait(b):
            pltpu.make_async_copy(pk_hbm.at[pl.ds(0, CH)], pkst.at[b], sis[b]).wait()
            pltpu.make_async_copy(norm_hbm.at[pl.ds(0, CH)], nst.at[b], sis[b]).wait()

        HF = CH // 2

        def gs_compute(b, base_m):
            for g in range(CH // LN):
                pv = pkst[b, pl.ds(g * LN, LN)]
                gidx[b, pl.ds(g * LN, LN)] = (pv >> 14) * stride + base_m
                scidx[2 * b + g // 4, pl.ds((g % 4) * LN, LN)] = pv & 16383

        def gather_start(b):
            pltpu.async_copy(src_hbm.at[gidx.at[b]], pays[b], sps[b])

        def gather_wait(b):
            pltpu.make_async_copy(src_hbm.at[gidx.at[b]], pays[b], sps[b]).wait()

        def scale_half(b, h):
            pay = pays[b]
            @pl.loop(0, CH // LN // 2)
            def _grp(g0):
                g = h * (CH // LN // 2) + g0
                nv = nst[b, pl.ds(g * LN, LN)]
                for i in range(LN):
                    for j in range(D // LN):
                        v = pay[g * LN + i, pl.ds(j * LN, LN)]
                        pay[g * LN + i, pl.ds(j * LN, LN)] = v * nv[i]

        def scatter_half(b, h):
            pltpu.async_copy(pays[b].at[pl.ds(h * HF, HF)],
                             acc_sh.at[scidx.at[2 * b + h]], sss[b], add=True)

        def scale(b):
            scale_half(b, 0)
            scatter_half(b, 0)
            scale_half(b, 1)

        def scatter_start(b):
            scatter_half(b, 1)

        def scatter_wait(b):
            for h in (0, 1):
                pltpu.make_async_copy(pays[b].at[pl.ds(h * HF, HF)],
                                      acc_sh.at[scidx.at[2 * b + h]],
                                      sss[b]).wait()

        nfull = N // NPT
        rem = N - nfull * NPT

        def copyout_start(m):
            if rem == 0:
                pltpu.async_copy(acc_sh.at[pl.ds(my0, NPT)],
                                 dst_hbm.at[m].at[pl.ds(my0, NPT)], so)
            else:
                @pl.when(sid < nfull)
                def _full():
                    pltpu.async_copy(acc_sh.at[pl.ds(my0, NPT)],
                                     dst_hbm.at[m].at[pl.ds(my0, NPT)], so)

                @pl.when(sid == nfull)
                def _tail():
                    pltpu.async_copy(acc_sh.at[pl.ds(my0, rem)],
                                     dst_hbm.at[m].at[pl.ds(my0, rem)], so)

        def copyout_wait():
            if rem == 0:
                pltpu.make_async_copy(acc_sh.at[pl.ds(my0, NPT)],
                                      dst_hbm.at[0].at[pl.ds(my0, NPT)], so).wait()
            else:
                @pl.when(sid < nfull)
                def _full():
                    pltpu.make_async_copy(acc_sh.at[pl.ds(my0, NPT)],
                                          dst_hbm.at[0].at[pl.ds(my0, NPT)], so).wait()

                @pl.when(sid == nfull)
                def _tail():
                    pltpu.make_async_copy(acc_sh.at[pl.ds(my0, rem)],
                                          dst_hbm.at[0].at[pl.ds(my0, rem)], so).wait()

        @pl.loop(0, MC)
        def _inst(inst):
            m = inst * NC + cid
            base_m = base_of_m(m)

            idx_start(0, 0)
            idx_start(1, 1)

            @pl.when(inst > 0)
            def _():
                copyout_wait()

            for z in range(NPT // ZR):
                pltpu.async_copy(zbuf, acc_sh.at[pl.ds(my0 + z * ZR, ZR)], sz)
            if NPT % ZR:
                pltpu.async_copy(zbuf.at[pl.ds(0, NPT % ZR)],
                                 acc_sh.at[pl.ds(my0 + (NPT // ZR) * ZR, NPT % ZR)], sz)

            idx_wait(0)
            gs_compute(0, base_m)
            gather_start(0)
            idx_wait(1)
            gs_compute(1, base_m)
            gather_start(1)

            for z in range(NPT // ZR):
                pltpu.make_async_copy(zbuf, acc_sh.at[pl.ds(my0, ZR)], sz).wait()
            if NPT % ZR:
                pltpu.make_async_copy(zbuf.at[pl.ds(0, NPT % ZR)],
                                      acc_sh.at[pl.ds(my0, NPT % ZR)], sz).wait()
            plsc.subcore_barrier()

            gather_wait(0)
            scale(0)
            idx_start(0, 2)
            scatter_start(0)

            @pl.loop(0, (NCH - 2) // 2)
            def _pair(p):
                for b, cc in ((1, 2 * p + 1), (0, 2 * p + 2)):
                    nb = 1 - b
                    idx_wait(nb)
                    scatter_wait(nb)
                    gs_compute(nb, base_m)
                    gather_start(nb)
                    gather_wait(b)
                    scale(b)
                    if b == 1:
                        idx_start(b, 2 * p + 3)
                    else:
                        @pl.when(2 * p + 4 < NCH)
                        def _():
                            idx_start(0, 2 * p + 4)
                    scatter_start(b)

            gather_wait(1)
            scale(1)
            scatter_start(1)
            scatter_wait(0)
            scatter_wait(1)

            plsc.subcore_barrier()

            copyout_start(m)

        copyout_wait()

    kern = pl.kernel(
        body,
        out_type=jax.ShapeDtypeStruct((M, N, D), jnp.float32),
        mesh=mesh,
        compiler_params=pltpu.CompilerParams(needs_layout_passes=False),
        scratch_types=[
            pltpu.VMEM_SHARED((NP, D), jnp.float32),
            pltpu.VMEM((ZR, D), jnp.float32),
            pltpu.VMEM((2, CH), jnp.int32),
            pltpu.VMEM((2, CH), jnp.float32),
            pltpu.VMEM((2, CH), jnp.int32),
            pltpu.VMEM((4, CH // 2), jnp.int32),
            pltpu.VMEM((CH, D), jnp.float32),
            pltpu.VMEM((CH, D), jnp.float32),
            pltpu.SemaphoreType.DMA,
            pltpu.SemaphoreType.DMA,
            pltpu.SemaphoreType.DMA,
            pltpu.SemaphoreType.DMA,
            pltpu.SemaphoreType.DMA,
            pltpu.SemaphoreType.DMA,
            pltpu.SemaphoreType.DMA,
            pltpu.SemaphoreType.DMA,
        ],
    )
    return kern


def _tmp_body(x_ref, e2_ref, out_ref):
    nb = pl.program_id(1)

    @pl.when(nb == 0)
    def _():
        out_ref[...] = jnp.zeros_like(out_ref)

    xb = x_ref[0]
    e2b = e2_ref[...]
    out_ref[0] += lax.dot_general(e2b, xb, (((0,), (0,)), ((), ())),
                                  preferred_element_type=jnp.float32)


def _fused_body(x_ref, t1_ref, p2_ref, tmp_ref, e1_ref,
                wa_ref, ba_ref, wc_ref, bc_ref, wfa_ref, bfa_ref,
                wfg_ref, bfg_ref, out_ref):
    xb = x_ref[0]
    t1b = t1_ref[0]
    p2b = p2_ref[0]
    tmpm = tmp_ref[0]
    e1b = e1_ref[...]
    dot = functools.partial(jnp.dot, preferred_element_type=jnp.float32)

    h = jax.nn.relu(dot(e1b, tmpm))
    agcn = dot(h, wa_ref[...]) + ba_ref[0]
    w0 = wc_ref[0]
    w1 = wc_ref[1]
    w2 = wc_ref[2]
    cheb = (dot(xb, w0 - w2) + dot(t1b, w1) + 2.0 * dot(p2b, w2) + bc_ref[0])
    gate = jax.nn.sigmoid(dot(agcn, wfa_ref[...]) + bfa_ref[0]
                          + dot(cheb, wfg_ref[...]) + bfg_ref[0])
    out_ref[0] = gate * agcn + (1.0 - gate) * cheb


def kernel(x, edge_index_distance, edge_weight_distance, E1, E2,
           W_agcn, b_agcn, W_cheb, b_cheb, Wf_a, bf_a, Wf_g, bf_g):
    B, N, T, D = x.shape
    M = B * T
    H = W_agcn.shape[1]
    R = E1.shape[1]
    E = edge_weight_distance.shape[0]
    RP = ((R + 7) // 8) * 8

    EUNIT = NS * CH * 8
    EPAD = ((E + EUNIT - 1) // EUNIT) * EUNIT
    padn = EPAD - E
    row = edge_index_distance[0].astype(jnp.int32)
    col = edge_index_distance[1].astype(jnp.int32)
    w = edge_weight_distance.astype(jnp.float32)
    if padn:
        spread = jnp.arange(padn, dtype=jnp.int32) % N
        row = jnp.concatenate([row, spread])
        col = jnp.concatenate([col, spread])
        w = jnp.concatenate([w, jnp.zeros((padn,), jnp.float32)])

    pk = (row << 14) | col
    norm = _make_norm_kernel(N, EPAD)(pk, w)
    xflat = x.reshape(B * N * T, D)
    prop1 = _make_prop_kernel(
        N, D, M, EPAD, B * N * T, T,
        lambda m: (m // T) * (N * T) + (m % T))
    tx1 = prop1(xflat, pk, norm)

    prop2 = _make_prop_kernel(
        N, D, M, EPAD, M * N, 1,
        lambda m: m * N)
    p2 = prop2(tx1.reshape(M * N, D), pk, norm)

    PB = 1000
    NBLK = N // PB
    E2p = jnp.zeros((N, RP), jnp.float32).at[:, :R].set(E2.T)
    E1p = jnp.zeros((N, RP), jnp.float32).at[:, :R].set(E1)

    xv = x.reshape(B, N, T * D)
    tmp = pl.pallas_call(
        _tmp_body,
        grid=(M, NBLK),
        in_specs=[
            pl.BlockSpec((1, PB, D), lambda m, nb: (m // T, nb, m % T)),
            pl.BlockSpec((PB, RP), lambda m, nb: (nb, 0)),
        ],
        out_specs=pl.BlockSpec((1, RP, D), lambda m, nb: (m, 0, 0)),
        out_shape=jax.ShapeDtypeStruct((M, RP, D), jnp.float32),
    )(xv, E2p)

    full2 = lambda a, b: pl.BlockSpec((a, b), lambda m, nb: (0, 0))
    out = pl.pallas_call(
        _fused_body,
        grid=(M, NBLK),
        in_specs=[
            pl.BlockSpec((1, PB, D), lambda m, nb: (m // T, nb, m % T)),
            pl.BlockSpec((1, PB, D), lambda m, nb: (m, nb, 0)),
            pl.BlockSpec((1, PB, D), lambda m, nb: (m, nb, 0)),
            pl.BlockSpec((1, RP, D), lambda m, nb: (m, 0, 0)),
            pl.BlockSpec((PB, RP), lambda m, nb: (nb, 0)),
            full2(D, H),
            full2(1, H),
            pl.BlockSpec((3, D, H), lambda m, nb: (0, 0, 0)),
            full2(1, H),
            full2(H, H),
            full2(1, H),
            full2(H, H),
            full2(1, H),
        ],
        out_specs=pl.BlockSpec((1, PB, H), lambda m, nb: (m // T, nb, m % T)),
        out_shape=jax.ShapeDtypeStruct((B, N, T * H), jnp.float32),
    )(xv, tx1, p2, tmp, E1p,
      W_agcn, b_agcn.reshape(1, H), W_cheb, b_cheb.reshape(1, H),
      Wf_a, bf_a.reshape(1, H), Wf_g, bf_g.reshape(1, H))
    return out.reshape(B, N, T, H)
